# Initial kernel scaffold; baseline (speedup 1.0000x reference)
import jax
import jax.numpy as jnp
from jax.experimental import pallas as pl


def kernel(x, weight, label):
    raise NotImplementedError("write your pallas kernel here")


# speedup vs baseline: 2.8714x; 2.8714x over previous
import functools
import math

import jax
import jax.numpy as jnp
from jax.experimental import pallas as pl
from jax.experimental.pallas import tpu as pltpu

_MARGIN = 0.3
_SCALE = 15.0
_COS_M = math.cos(_MARGIN)
_SIN_M = math.sin(_MARGIN)
_TH = math.cos(math.pi - _MARGIN)
_MM = math.sin(math.pi - _MARGIN) * _MARGIN
_EPS = 1e-12
_LN2 = math.log(2.0)
_S2 = _SCALE / _LN2
_NEG = -1e30


def _aam_body(nc, C, x_ref, w_ref, lab_ref, nll_ref, cor_ref,
              xn_ref, m_ref, l_ref, bi_ref, tl_ref):
    c = pl.program_id(1)
    BB = x_ref.shape[0]
    BC = w_ref.shape[0]

    @pl.when(c == 0)
    def _init():
        xs = x_ref[...]
        inv = 1.0 / jnp.maximum(
            jnp.sqrt(jnp.sum(xs * xs, axis=1, keepdims=True)), _EPS)
        xn_ref[...] = xs * inv
        m_ref[...] = jnp.full((BB, 1), _NEG, jnp.float32)
        l_ref[...] = jnp.zeros((BB, 1), jnp.float32)
        bi_ref[...] = jnp.zeros((BB, 1), jnp.int32)
        tl_ref[...] = jnp.zeros((BB, 1), jnp.float32)

    ws = w_ref[...]
    winv = 1.0 / jnp.maximum(
        jnp.sqrt(jnp.sum(ws * ws, axis=1, keepdims=True)), _EPS)
    wn = ws * winv
    cos = jax.lax.dot_general(xn_ref[...], wn, (((1,), (1,)), ((), ())),
                              preferred_element_type=jnp.float32)

    iota = jax.lax.broadcasted_iota(jnp.int32, (BB, BC), 1)
    col0 = c * BC
    valid = (iota + col0) < C
    l2s = jnp.where(valid, cos * _S2, _NEG)

    local = lab_ref[...] - col0
    onehot = iota == local
    has_t = (local >= 0) & (local < BC)
    tcos = jnp.sum(jnp.where(onehot, cos, 0.0), axis=1, keepdims=True)
    sine = jnp.sqrt(jnp.clip(1.0 - tcos * tcos, 0.0, 1.0))
    phi = tcos * _COS_M - sine * _SIN_M
    phi = jnp.where(tcos - _TH > 0, phi, tcos - _MM)
    tval = phi * _S2
    l2s = jnp.where(onehot, tval, l2s)
    tl_ref[...] = tl_ref[...] + jnp.where(has_t, tval, 0.0)

    mb = jnp.max(l2s, axis=1, keepdims=True)
    bi_loc = jnp.min(jnp.where(l2s == mb, iota, BC), axis=1, keepdims=True)
    m_old = m_ref[...]
    upd = mb > m_old
    m_new = jnp.where(upd, mb, m_old)
    bi_ref[...] = jnp.where(upd, bi_loc + col0, bi_ref[...])
    p = jnp.exp2(l2s - m_new)
    sb = jnp.sum(p, axis=1, keepdims=True)
    l_ref[...] = l_ref[...] * jnp.exp2(m_old - m_new) + sb
    m_ref[...] = m_new

    @pl.when(c == nc - 1)
    def _fin():
        lse2 = m_ref[...] + jnp.log2(l_ref[...])
        nll_ref[...] = (lse2 - tl_ref[...]) * _LN2
        cor_ref[...] = (bi_ref[...] == lab_ref[...]).astype(jnp.float32)


def kernel(x, weight, label):
    B, D = x.shape
    C = weight.shape[0]
    BB = 1024
    BC = 2048
    nb = B // BB
    nc = (C + BC - 1) // BC
    lab = label.astype(jnp.int32).reshape(B, 1)
    nll, cor = pl.pallas_call(
        functools.partial(_aam_body, nc, C),
        grid=(nb, nc),
        in_specs=[
            pl.BlockSpec((BB, D), lambda b, c: (b, 0)),
            pl.BlockSpec((BC, D), lambda b, c: (c, 0)),
            pl.BlockSpec((BB, 1), lambda b, c: (b, 0)),
        ],
        out_specs=[
            pl.BlockSpec((BB, 1), lambda b, c: (b, 0)),
            pl.BlockSpec((BB, 1), lambda b, c: (b, 0)),
        ],
        out_shape=[
            jax.ShapeDtypeStruct((B, 1), jnp.float32),
            jax.ShapeDtypeStruct((B, 1), jnp.float32),
        ],
        scratch_shapes=[
            pltpu.VMEM((BB, D), jnp.float32),
            pltpu.VMEM((BB, 1), jnp.float32),
            pltpu.VMEM((BB, 1), jnp.float32),
            pltpu.VMEM((BB, 1), jnp.int32),
            pltpu.VMEM((BB, 1), jnp.float32),
        ],
        compiler_params=pltpu.CompilerParams(
            dimension_semantics=("parallel", "arbitrary"),
            vmem_limit_bytes=48 * 1024 * 1024,
        ),
        name="aamsoftmax_fused",
    )(x, weight, lab)
    loss = jnp.mean(nll[:, 0])
    prec1 = jnp.mean(cor[:, 0]) * 100.0
    return (loss, prec1)


# speedup vs baseline: 3.8554x; 1.3427x over previous
import functools
import math

import jax
import jax.numpy as jnp
from jax.experimental import pallas as pl
from jax.experimental.pallas import tpu as pltpu

_MARGIN = 0.3
_SCALE = 15.0
_COS_M = math.cos(_MARGIN)
_SIN_M = math.sin(_MARGIN)
_TH = math.cos(math.pi - _MARGIN)
_MM = math.sin(math.pi - _MARGIN) * _MARGIN
_EPS = 1e-12
_LN2 = math.log(2.0)
_S2 = _SCALE / _LN2
_NEG = -1e30


def _aam_body(nc, x_ref, w_ref, lab_ref, nll_ref, cor_ref,
              xn_ref, m_ref, l_ref, tl_ref):
    c = pl.program_id(1)
    BB = x_ref.shape[0]
    BC = w_ref.shape[0]

    @pl.when(c == 0)
    def _init():
        xs = x_ref[...]
        inv = _S2 / jnp.maximum(
            jnp.sqrt(jnp.sum(xs * xs, axis=1, keepdims=True)), _EPS)
        xn_ref[...] = xs * inv
        m_ref[...] = jnp.full((BB, 1), _NEG, jnp.float32)
        l_ref[...] = jnp.zeros((BB, 1), jnp.float32)
        tl_ref[...] = jnp.zeros((BB, 1), jnp.float32)

    ws = w_ref[...]
    winv = 1.0 / jnp.maximum(
        jnp.sqrt(jnp.sum(ws * ws, axis=1, keepdims=True)), _EPS)
    wn = ws * winv
    l2s = jax.lax.dot_general(xn_ref[...], wn, (((1,), (1,)), ((), ())),
                              preferred_element_type=jnp.float32)

    local = lab_ref[...] - c * BC
    iota = jax.lax.broadcasted_iota(jnp.int32, (BB, BC), 1)
    onehot = iota == local
    nlv = jnp.where(onehot, _NEG, l2s)
    tl_ref[...] = tl_ref[...] + jnp.sum(
        jnp.where(onehot, l2s, 0.0), axis=1, keepdims=True)

    mb = jnp.max(nlv, axis=1, keepdims=True)
    m_old = m_ref[...]
    m_new = jnp.maximum(m_old, mb)
    p = jnp.exp2(nlv - m_new)
    sb = jnp.sum(p, axis=1, keepdims=True)
    l_ref[...] = l_ref[...] * jnp.exp2(m_old - m_new) + sb
    m_ref[...] = m_new

    @pl.when(c == nc - 1)
    def _fin():
        m = m_ref[...]
        tcos = tl_ref[...] * (_LN2 / _SCALE)
        sine = jnp.sqrt(jnp.clip(1.0 - tcos * tcos, 0.0, 1.0))
        phi = tcos * _COS_M - sine * _SIN_M
        phi = jnp.where(tcos - _TH > 0, phi, tcos - _MM)
        tv = phi * _S2
        nll_ref[...] = (m + jnp.log2(l_ref[...] + jnp.exp2(tv - m)) - tv) * _LN2
        cor_ref[...] = (tv > m).astype(jnp.float32)


def kernel(x, weight, label):
    B, D = x.shape
    C = weight.shape[0]
    BB = 1024
    BC = 2000
    nb = B // BB
    nc = C // BC
    lab = label.astype(jnp.int32).reshape(B, 1)
    nll, cor = pl.pallas_call(
        functools.partial(_aam_body, nc),
        grid=(nb, nc),
        in_specs=[
            pl.BlockSpec((BB, D), lambda b, c: (b, 0)),
            pl.BlockSpec((BC, D), lambda b, c: (c, 0)),
            pl.BlockSpec((BB, 1), lambda b, c: (b, 0)),
        ],
        out_specs=[
            pl.BlockSpec((BB, 1), lambda b, c: (b, 0)),
            pl.BlockSpec((BB, 1), lambda b, c: (b, 0)),
        ],
        out_shape=[
            jax.ShapeDtypeStruct((B, 1), jnp.float32),
            jax.ShapeDtypeStruct((B, 1), jnp.float32),
        ],
        scratch_shapes=[
            pltpu.VMEM((BB, D), jnp.float32),
            pltpu.VMEM((BB, 1), jnp.float32),
            pltpu.VMEM((BB, 1), jnp.float32),
            pltpu.VMEM((BB, 1), jnp.float32),
        ],
        compiler_params=pltpu.CompilerParams(
            dimension_semantics=("parallel", "arbitrary"),
            vmem_limit_bytes=48 * 1024 * 1024,
        ),
        name="aamsoftmax_fused",
    )(x, weight, lab)
    loss = jnp.mean(nll[:, 0])
    prec1 = jnp.mean(cor[:, 0]) * 100.0
    return (loss, prec1)


# speedup vs baseline: 3.8648x; 1.0024x over previous
import functools
import math

import jax
import jax.numpy as jnp
from jax.experimental import pallas as pl
from jax.experimental.pallas import tpu as pltpu

_MARGIN = 0.3
_SCALE = 15.0
_COS_M = math.cos(_MARGIN)
_SIN_M = math.sin(_MARGIN)
_TH = math.cos(math.pi - _MARGIN)
_MM = math.sin(math.pi - _MARGIN) * _MARGIN
_EPS = 1e-12
_LN2 = math.log(2.0)
_S2 = _SCALE / _LN2
_NEG = -1e30
_SUB = 80


def _aam_body(nc, xt_ref, w_ref, lab_ref, nll_ref, cor_ref,
              xnt_ref, m_ref, l_ref, tl_ref):
    c = pl.program_id(1)
    BB = xt_ref.shape[1]
    BC = w_ref.shape[0]

    @pl.when(c == 0)
    def _init():
        xt = xt_ref[...]
        n2 = jnp.sum(xt * xt, axis=0, keepdims=True)
        inv = _S2 / jnp.maximum(jnp.sqrt(n2), _EPS)
        xnt_ref[...] = xt * inv
        m_ref[...] = jnp.full((1, BB), _NEG, jnp.float32)
        l_ref[...] = jnp.zeros((1, BB), jnp.float32)
        tl_ref[...] = jnp.zeros((1, BB), jnp.float32)

    ws = w_ref[...]
    winv = 1.0 / jnp.maximum(
        jnp.sqrt(jnp.sum(ws * ws, axis=1, keepdims=True)), _EPS)
    wn = ws * winv
    l2s = jax.lax.dot_general(wn, xnt_ref[...], (((1,), (0,)), ((), ())),
                              preferred_element_type=jnp.float32)

    loc = lab_ref[0] - c * BC
    m_run = m_ref[...]
    l_run = l_ref[...]
    tl_run = tl_ref[...]
    for s in range(BC // _SUB):
        blk = l2s[s * _SUB:(s + 1) * _SUB, :]
        ci = jax.lax.broadcasted_iota(jnp.int32, (_SUB, 1), 0) + (s * _SUB)
        oh = ci == loc
        nlv = jnp.where(oh, _NEG, blk)
        tl_run = tl_run + jnp.sum(jnp.where(oh, blk, 0.0),
                                  axis=0, keepdims=True)
        mb = jnp.max(nlv, axis=0, keepdims=True)
        m_new = jnp.maximum(m_run, mb)
        p = jnp.exp2(nlv - m_new)
        l_run = (l_run * jnp.exp2(m_run - m_new)
                 + jnp.sum(p, axis=0, keepdims=True))
        m_run = m_new
    m_ref[...] = m_run
    l_ref[...] = l_run
    tl_ref[...] = tl_run

    @pl.when(c == nc - 1)
    def _fin():
        tcos = tl_run * (_LN2 / _SCALE)
        sine = jnp.sqrt(jnp.clip(1.0 - tcos * tcos, 0.0, 1.0))
        phi = tcos * _COS_M - sine * _SIN_M
        phi = jnp.where(tcos - _TH > 0, phi, tcos - _MM)
        tv = phi * _S2
        nll = (m_run + jnp.log2(l_run + jnp.exp2(tv - m_run)) - tv) * _LN2
        nll_ref[...] = nll.reshape(1, 1, BB)
        cor_ref[...] = (tv > m_run).astype(jnp.float32).reshape(1, 1, BB)


def kernel(x, weight, label):
    B, D = x.shape
    C = weight.shape[0]
    BB = 1024
    BC = 2000
    nb = B // BB
    nc = C // BC
    xt = x.T
    lab = label.astype(jnp.int32).reshape(nb, 1, BB)
    nll, cor = pl.pallas_call(
        functools.partial(_aam_body, nc),
        grid=(nb, nc),
        in_specs=[
            pl.BlockSpec((D, BB), lambda b, c: (0, b)),
            pl.BlockSpec((BC, D), lambda b, c: (c, 0)),
            pl.BlockSpec((1, 1, BB), lambda b, c: (b, 0, 0)),
        ],
        out_specs=[
            pl.BlockSpec((1, 1, BB), lambda b, c: (b, 0, 0)),
            pl.BlockSpec((1, 1, BB), lambda b, c: (b, 0, 0)),
        ],
        out_shape=[
            jax.ShapeDtypeStruct((nb, 1, BB), jnp.float32),
            jax.ShapeDtypeStruct((nb, 1, BB), jnp.float32),
        ],
        scratch_shapes=[
            pltpu.VMEM((D, BB), jnp.float32),
            pltpu.VMEM((1, BB), jnp.float32),
            pltpu.VMEM((1, BB), jnp.float32),
            pltpu.VMEM((1, BB), jnp.float32),
        ],
        compiler_params=pltpu.CompilerParams(
            dimension_semantics=("parallel", "arbitrary"),
            vmem_limit_bytes=48 * 1024 * 1024,
        ),
        name="aamsoftmax_fused",
    )(xt, weight, lab)
    loss = jnp.mean(nll)
    prec1 = jnp.mean(cor) * 100.0
    return (loss, prec1)


# speedup vs baseline: 4.0119x; 1.0381x over previous
import functools
import math

import jax
import jax.numpy as jnp
from jax.experimental import pallas as pl
from jax.experimental.pallas import tpu as pltpu

_MARGIN = 0.3
_SCALE = 15.0
_COS_M = math.cos(_MARGIN)
_SIN_M = math.sin(_MARGIN)
_TH = math.cos(math.pi - _MARGIN)
_MM = math.sin(math.pi - _MARGIN) * _MARGIN
_EPS = 1e-12
_LN2 = math.log(2.0)
_S2 = _SCALE / _LN2
_NEG = -1e30
_SUB = 80


def _aam_body(nc, xt_ref, w_ref, lab_ref, nll_ref, cor_ref,
              xnt_ref, m_ref, l_ref, tl_ref):
    c = pl.program_id(0)
    BB = xt_ref.shape[1]
    BC = w_ref.shape[0]

    @pl.when(c == 0)
    def _init():
        xt = xt_ref[...]
        n2 = jnp.sum(xt * xt, axis=0, keepdims=True)
        inv = _S2 / jnp.maximum(jnp.sqrt(n2), _EPS)
        xnt_ref[...] = xt * inv
        m_ref[...] = jnp.full((1, BB), _NEG, jnp.float32)
        l_ref[...] = jnp.zeros((1, BB), jnp.float32)
        tl_ref[...] = jnp.zeros((1, BB), jnp.float32)

    ws = w_ref[...]
    winv = 1.0 / jnp.maximum(
        jnp.sqrt(jnp.sum(ws * ws, axis=1, keepdims=True)), _EPS)
    wn = ws * winv
    l2s = jax.lax.dot_general(wn, xnt_ref[...], (((1,), (0,)), ((), ())),
                              preferred_element_type=jnp.float32)

    loc = lab_ref[0] - c * BC
    m_run = m_ref[...]
    l_run = l_ref[...]
    tl_run = tl_ref[...]
    for s in range(BC // _SUB):
        blk = l2s[s * _SUB:(s + 1) * _SUB, :]
        ci = jax.lax.broadcasted_iota(jnp.int32, (_SUB, 1), 0) + (s * _SUB)
        oh = ci == loc
        nlv = jnp.where(oh, _NEG, blk)
        tl_run = tl_run + jnp.sum(jnp.where(oh, blk, 0.0),
                                  axis=0, keepdims=True)
        mb = jnp.max(nlv, axis=0, keepdims=True)
        m_new = jnp.maximum(m_run, mb)
        p = jnp.exp2(nlv - m_new)
        l_run = (l_run * jnp.exp2(m_run - m_new)
                 + jnp.sum(p, axis=0, keepdims=True))
        m_run = m_new
    m_ref[...] = m_run
    l_ref[...] = l_run
    tl_ref[...] = tl_run

    @pl.when(c == nc - 1)
    def _fin():
        tcos = tl_run * (_LN2 / _SCALE)
        sine = jnp.sqrt(jnp.clip(1.0 - tcos * tcos, 0.0, 1.0))
        phi = tcos * _COS_M - sine * _SIN_M
        phi = jnp.where(tcos - _TH > 0, phi, tcos - _MM)
        tv = phi * _S2
        nll = (m_run + jnp.log2(l_run + jnp.exp2(tv - m_run)) - tv) * _LN2
        nll_ref[...] = nll.reshape(1, 1, BB)
        cor_ref[...] = (tv > m_run).astype(jnp.float32).reshape(1, 1, BB)


def kernel(x, weight, label):
    B, D = x.shape
    C = weight.shape[0]
    BB = B
    BC = 2000
    nc = C // BC
    xt = x.T
    lab = label.astype(jnp.int32).reshape(1, 1, BB)
    nll, cor = pl.pallas_call(
        functools.partial(_aam_body, nc),
        grid=(nc,),
        in_specs=[
            pl.BlockSpec((D, BB), lambda c: (0, 0)),
            pl.BlockSpec((BC, D), lambda c: (c, 0)),
            pl.BlockSpec((1, 1, BB), lambda c: (0, 0, 0)),
        ],
        out_specs=[
            pl.BlockSpec((1, 1, BB), lambda c: (0, 0, 0)),
            pl.BlockSpec((1, 1, BB), lambda c: (0, 0, 0)),
        ],
        out_shape=[
            jax.ShapeDtypeStruct((1, 1, BB), jnp.float32),
            jax.ShapeDtypeStruct((1, 1, BB), jnp.float32),
        ],
        scratch_shapes=[
            pltpu.VMEM((D, BB), jnp.float32),
            pltpu.VMEM((1, BB), jnp.float32),
            pltpu.VMEM((1, BB), jnp.float32),
            pltpu.VMEM((1, BB), jnp.float32),
        ],
        compiler_params=pltpu.CompilerParams(
            dimension_semantics=("arbitrary",),
            vmem_limit_bytes=56 * 1024 * 1024,
        ),
        name="aamsoftmax_fused",
    )(xt, weight, lab)
    loss = jnp.mean(nll)
    prec1 = jnp.mean(cor) * 100.0
    return (loss, prec1)


# speedup vs baseline: 5.3451x; 1.3323x over previous
import functools
import math

import jax
import jax.numpy as jnp
from jax.experimental import pallas as pl
from jax.experimental.pallas import tpu as pltpu

_MARGIN = 0.3
_SCALE = 15.0
_COS_M = math.cos(_MARGIN)
_SIN_M = math.sin(_MARGIN)
_TH = math.cos(math.pi - _MARGIN)
_MM = math.sin(math.pi - _MARGIN) * _MARGIN
_EPS = 1e-12
_LN2 = math.log(2.0)
_S2 = _SCALE / _LN2
_NEG = -1e30
_SUB = 80


def _tree(parts, op):
    while len(parts) > 1:
        nxt = [op(parts[i], parts[i + 1]) for i in range(0, len(parts) - 1, 2)]
        if len(parts) % 2:
            nxt.append(parts[-1])
        parts = nxt
    return parts[0]


def _to8(a, op):
    return _tree([a[8 * r:8 * (r + 1)] for r in range(a.shape[0] // 8)], op)


def _aam_body(nc, xt_ref, w_ref, lab_ref, nll_ref, cor_ref,
              xnt_ref, m8_ref, l8_ref, t8_ref):
    c = pl.program_id(0)
    BB = xt_ref.shape[1]
    BC = w_ref.shape[0]

    @pl.when(c == 0)
    def _init():
        xt = xt_ref[...]
        n2 = jnp.sum(xt * xt, axis=0, keepdims=True)
        inv = _S2 / jnp.maximum(jnp.sqrt(n2), _EPS)
        xnt_ref[...] = xt * inv
        m8_ref[...] = jnp.full((8, BB), _NEG, jnp.float32)
        l8_ref[...] = jnp.zeros((8, BB), jnp.float32)
        t8_ref[...] = jnp.zeros((8, BB), jnp.float32)

    ws = w_ref[...]
    winv = 1.0 / jnp.maximum(
        jnp.sqrt(jnp.sum(ws * ws, axis=1, keepdims=True)), _EPS)
    wn = ws * winv
    l2s = jax.lax.dot_general(wn, xnt_ref[...], (((1,), (0,)), ((), ())),
                              preferred_element_type=jnp.float32)

    loc = lab_ref[0] - c * BC
    m8 = m8_ref[...]
    l8 = l8_ref[...]
    t8 = t8_ref[...]
    add = lambda a, b: a + b
    vmax = jnp.maximum
    for s in range(BC // _SUB):
        blk = l2s[s * _SUB:(s + 1) * _SUB, :]
        ci = jax.lax.broadcasted_iota(jnp.int32, (_SUB, 1), 0) + (s * _SUB)
        oh = ci == loc
        l8 = l8 + _to8(jnp.exp2(blk), add)
        m8 = vmax(m8, _to8(jnp.where(oh, _NEG, blk), vmax))
        t8 = t8 + _to8(jnp.where(oh, blk, 0.0), add)
    m8_ref[...] = m8
    l8_ref[...] = l8
    t8_ref[...] = t8

    @pl.when(c == nc - 1)
    def _fin():
        m = jnp.max(m8, axis=0, keepdims=True)
        l_all = jnp.sum(l8, axis=0, keepdims=True)
        tl = jnp.sum(t8, axis=0, keepdims=True)
        tcos = tl * (_LN2 / _SCALE)
        sine = jnp.sqrt(jnp.clip(1.0 - tcos * tcos, 0.0, 1.0))
        phi = tcos * _COS_M - sine * _SIN_M
        phi = jnp.where(tcos - _TH > 0, phi, tcos - _MM)
        tv = phi * _S2
        l_adj = jnp.maximum(l_all - jnp.exp2(tl) + jnp.exp2(tv), 1e-35)
        nll = (jnp.log2(l_adj) - tv) * _LN2
        nll_ref[...] = nll.reshape(1, 1, BB)
        cor_ref[...] = (tv > m).astype(jnp.float32).reshape(1, 1, BB)


def kernel(x, weight, label):
    B, D = x.shape
    C = weight.shape[0]
    BB = B
    BC = 2000
    nc = C // BC
    xt = x.T
    lab = label.astype(jnp.int32).reshape(1, 1, BB)
    nll, cor = pl.pallas_call(
        functools.partial(_aam_body, nc),
        grid=(nc,),
        in_specs=[
            pl.BlockSpec((D, BB), lambda c: (0, 0)),
            pl.BlockSpec((BC, D), lambda c: (c, 0)),
            pl.BlockSpec((1, 1, BB), lambda c: (0, 0, 0)),
        ],
        out_specs=[
            pl.BlockSpec((1, 1, BB), lambda c: (0, 0, 0)),
            pl.BlockSpec((1, 1, BB), lambda c: (0, 0, 0)),
        ],
        out_shape=[
            jax.ShapeDtypeStruct((1, 1, BB), jnp.float32),
            jax.ShapeDtypeStruct((1, 1, BB), jnp.float32),
        ],
        scratch_shapes=[
            pltpu.VMEM((D, BB), jnp.float32),
            pltpu.VMEM((8, BB), jnp.float32),
            pltpu.VMEM((8, BB), jnp.float32),
            pltpu.VMEM((8, BB), jnp.float32),
        ],
        compiler_params=pltpu.CompilerParams(
            dimension_semantics=("arbitrary",),
            vmem_limit_bytes=56 * 1024 * 1024,
        ),
        name="aamsoftmax_fused",
    )(xt, weight, lab)
    loss = jnp.mean(nll)
    prec1 = jnp.mean(cor) * 100.0
    return (loss, prec1)
